# baseline (speedup 1.0000x reference)
import functools

import jax
import jax.numpy as jnp
from jax import lax
from jax.experimental import pallas as pl
from jax.experimental.pallas import tpu as pltpu
from jax.experimental.pallas import tpu_sc as plsc

_NC = 2
_NS = 16
_NW = _NC * _NS
_K = 8
_WB = 80
_G = 64


def _segsum_sc(N, D, nch, ch, pre):
    wb = _WB if D <= 32 else 40
    k = _K
    nwb = N // wb
    assert nwb * wb == N and D % 16 == 0 and nch % k == 0

    mesh = plsc.VectorSubcoreMesh(core_axis_name="c", subcore_axis_name="s",
                                  num_cores=_NC, num_subcores=_NS)
    out_type = [jax.ShapeDtypeStruct((_NC, N, D), jnp.float32)]
    if pre:
        out_type = [jax.ShapeDtypeStruct((_NC, N, D), jnp.float32)] + out_type
    scratch = [
        pltpu.VMEM((nch, ch), jnp.int32),
        pltpu.VMEM((nch, ch), jnp.int32),
        pltpu.VMEM((k, ch, D), jnp.float32),
        pltpu.VMEM((wb, D), jnp.float32),
        pltpu.VMEM_SHARED((N + 8, D), jnp.float32),
        pltpu.SemaphoreType.DMA,
        pltpu.SemaphoreType.DMA,
    ]
    if pre:
        scratch.insert(4, pltpu.VMEM((2, D), jnp.float32))

    @functools.partial(
        pl.kernel,
        out_type=out_type,
        mesh=mesh,
        compiler_params=pltpu.CompilerParams(use_tc_tiling_on_sc=False),
        scratch_types=scratch,
    )
    def seg(*refs):
        if pre:
            (h_hbm, ab_hbm, src_hbm, dst_hbm, x_hbm, out_hbm,
             src_v, dst_v, bufs, zbuf, ab_v, acc, gsem, ssem) = refs
        else:
            (y_hbm, src_hbm, dst_hbm, out_hbm,
             src_v, dst_v, bufs, zbuf, acc, gsem, ssem) = refs
        c = lax.axis_index("c")
        s = lax.axis_index("s")
        wid = c * _NS + s
        pltpu.sync_copy(src_hbm.at[wid], src_v)
        pltpu.sync_copy(dst_hbm.at[wid], dst_v)

        nrounds = pl.cdiv(nwb, _NS)

        if pre:
            pltpu.sync_copy(ab_hbm, ab_v)

            def achunk(kk, carry):
                chunk = s + kk * _NS

                @pl.when(chunk < nwb)
                def _():
                    pltpu.sync_copy(h_hbm.at[pl.ds(chunk * wb, wb)], zbuf)

                    def arow(r, c2):
                        for t in range(D // 16):
                            sl = pl.ds(t * 16, 16)
                            zbuf[r, sl] = jnp.maximum(
                                zbuf[r, sl] * ab_v[0, sl] + ab_v[1, sl], 0.0)
                        return c2

                    lax.fori_loop(0, wb, arow, 0)
                    pltpu.sync_copy(zbuf,
                                    x_hbm.at[c].at[pl.ds(chunk * wb, wb)])

                return carry

            lax.fori_loop(0, nrounds, achunk, 0)
            gather_src = x_hbm.at[c]
        else:
            gather_src = y_hbm

        zvec = jnp.zeros((16,), jnp.float32)

        def zrow(i, carry):
            for t in range(D // 16):
                zbuf[i, pl.ds(t * 16, 16)] = zvec
            return carry

        lax.fori_loop(0, wb, zrow, 0)

        def zchunk(kk, carry):
            chunk = s + kk * _NS

            @pl.when(chunk < nwb)
            def _():
                pltpu.sync_copy(zbuf, acc.at[pl.ds(chunk * wb, wb)])

            return carry

        lax.fori_loop(0, nrounds, zchunk, 0)
        plsc.subcore_barrier()

        def group(g, carry):
            gath = []
            for b in range(k):
                j = g * k + b
                gath.append(pltpu.async_copy(
                    gather_src.at[src_v.at[j]], bufs.at[b], gsem))
            for h in gath:
                h.wait()
            scat = []
            for b in range(k):
                j = g * k + b
                scat.append(pltpu.async_copy(
                    bufs.at[b], acc.at[dst_v.at[j]], ssem, add=True))
            for h in scat:
                h.wait()
            return carry

        lax.fori_loop(0, nch // k, group, 0)
        plsc.subcore_barrier()

        def wchunk(kk, carry):
            chunk = s + kk * _NS

            @pl.when(chunk < nwb)
            def _():
                pltpu.sync_copy(acc.at[pl.ds(chunk * wb, wb)],
                                out_hbm.at[c].at[pl.ds(chunk * wb, wb)])

            return carry

        lax.fori_loop(0, nrounds, wchunk, 0)

    return seg


def _conv1_post(x, p, eps, Wa, ba, Wb, bb, g, be, bn):
    N, F = x.shape
    D = Wb.shape[1]
    nblk = N // bn

    def body(x_ref, p_ref, eps_ref, wa_ref, ba_ref, wb_ref, bb_ref, g_ref,
             be_ref, h_ref, ss_ref, st_acc):
        i = pl.program_id(0)
        e = eps_ref[0, 0]
        pre = (1.0 + e) * x_ref[...] + jnp.sum(p_ref[...], axis=0)
        t = jax.nn.relu(jnp.dot(pre, wa_ref[...],
                                preferred_element_type=jnp.float32) + ba_ref[...])
        h = jnp.dot(t, wb_ref[...], preferred_element_type=jnp.float32) + bb_ref[...]
        h_ref[...] = h
        st = jnp.concatenate([jnp.sum(h, axis=0, keepdims=True),
                              jnp.sum(h * h, axis=0, keepdims=True)], axis=0)

        @pl.when(i == 0)
        def _():
            st_acc[...] = st

        @pl.when(i > 0)
        def _():
            st_acc[...] += st

        @pl.when(i == nblk - 1)
        def _():
            m = st_acc[0:1, :] * (1.0 / N)
            v = st_acc[1:2, :] * (1.0 / N) - m * m
            scale = lax.rsqrt(v + 1e-5) * g_ref[...]
            shift = be_ref[...] - m * scale
            ss_ref[...] = jnp.concatenate([scale, shift], axis=0)

    Da = Wa.shape[1]
    return pl.pallas_call(
        body,
        grid=(nblk,),
        in_specs=[pl.BlockSpec((bn, F), lambda i: (i, 0)),
                  pl.BlockSpec((2, bn, F), lambda i: (0, i, 0)),
                  pl.BlockSpec((1, 1), lambda i: (0, 0)),
                  pl.BlockSpec((F, Da), lambda i: (0, 0)),
                  pl.BlockSpec((1, Da), lambda i: (0, 0)),
                  pl.BlockSpec((Da, D), lambda i: (0, 0)),
                  pl.BlockSpec((1, D), lambda i: (0, 0)),
                  pl.BlockSpec((1, D), lambda i: (0, 0)),
                  pl.BlockSpec((1, D), lambda i: (0, 0))],
        out_specs=[pl.BlockSpec((bn, D), lambda i: (i, 0)),
                   pl.BlockSpec((2, D), lambda i: (0, 0))],
        out_shape=[jax.ShapeDtypeStruct((N, D), jnp.float32),
                   jax.ShapeDtypeStruct((2, D), jnp.float32)],
        scratch_shapes=[pltpu.VMEM((2, D), jnp.float32)],
    )(x, p, eps, Wa, ba, Wb, bb, g, be)


def _conv2_post(x1, p, eps, Wa, ba, Wb, bb, g, be, bn):
    N, D = x1.shape
    Do = Wb.shape[1]
    nblk = N // bn

    def body(x_ref, p_ref, eps_ref, wa_ref, ba_ref, wb_ref, bb_ref, g_ref,
             be_ref, h_ref, ss_ref, st_acc):
        i = pl.program_id(0)
        e = eps_ref[0, 0]
        pre = (1.0 + e) * x_ref[...] + jnp.sum(p_ref[...], axis=0)
        t = jax.nn.relu(jnp.dot(pre, wa_ref[...],
                                preferred_element_type=jnp.float32) + ba_ref[...])
        h = jnp.dot(t, wb_ref[...], preferred_element_type=jnp.float32) + bb_ref[...]
        h_ref[...] = h
        st = jnp.concatenate([jnp.sum(h, axis=0, keepdims=True),
                              jnp.sum(h * h, axis=0, keepdims=True)], axis=0)

        @pl.when(i == 0)
        def _():
            st_acc[...] = st

        @pl.when(i > 0)
        def _():
            st_acc[...] += st

        @pl.when(i == nblk - 1)
        def _():
            m = st_acc[0:1, :] * (1.0 / N)
            v = st_acc[1:2, :] * (1.0 / N) - m * m
            scale = lax.rsqrt(v + 1e-5) * g_ref[...]
            shift = be_ref[...] - m * scale
            ss_ref[...] = jnp.concatenate([scale, shift], axis=0)

    Da = Wa.shape[1]
    return pl.pallas_call(
        body,
        grid=(nblk,),
        in_specs=[pl.BlockSpec((bn, D), lambda i: (i, 0)),
                  pl.BlockSpec((2, bn, D), lambda i: (0, i, 0)),
                  pl.BlockSpec((1, 1), lambda i: (0, 0)),
                  pl.BlockSpec((D, Da), lambda i: (0, 0)),
                  pl.BlockSpec((1, Da), lambda i: (0, 0)),
                  pl.BlockSpec((Da, Do), lambda i: (0, 0)),
                  pl.BlockSpec((1, Do), lambda i: (0, 0)),
                  pl.BlockSpec((1, Do), lambda i: (0, 0)),
                  pl.BlockSpec((1, Do), lambda i: (0, 0))],
        out_specs=[pl.BlockSpec((bn, Do), lambda i: (i, 0)),
                   pl.BlockSpec((2, Do), lambda i: (0, 0))],
        out_shape=[jax.ShapeDtypeStruct((N, Do), jnp.float32),
                   jax.ShapeDtypeStruct((2, Do), jnp.float32)],
        scratch_shapes=[pltpu.VMEM((2, Do), jnp.float32)],
    )(x1, p, eps, Wa, ba, Wb, bb, g, be)


def _pool_head(h, ss, batchr, fcW1, fcb1, fcW2, fcb2, bn):
    N, D = h.shape
    nblk = N // bn

    def body(h_ref, ss_ref, b_ref, w1_ref, b1_ref, w2_ref, b2_ref,
             out_ref, pool_acc, cnt_acc):
        i = pl.program_id(0)
        x2 = jax.nn.relu(h_ref[...] * ss_ref[0:1, :] + ss_ref[1:2, :])
        oh = (lax.broadcasted_iota(jnp.int32, (_G, bn), 0)
              == b_ref[0]).astype(jnp.float32)
        pool_part = jnp.dot(oh, x2, preferred_element_type=jnp.float32,
                            precision=lax.Precision.HIGHEST)
        cnt_part = jnp.sum(oh, axis=1, keepdims=True)

        @pl.when(i == 0)
        def _():
            pool_acc[...] = pool_part
            cnt_acc[...] = cnt_part

        @pl.when(i > 0)
        def _():
            pool_acc[...] += pool_part
            cnt_acc[...] += cnt_part

        @pl.when(i == nblk - 1)
        def _():
            pool = pool_acc[...] / jnp.maximum(cnt_acc[...], 1.0)
            hh = jax.nn.relu(jnp.dot(pool, w1_ref[...],
                                     preferred_element_type=jnp.float32)
                             + b1_ref[...]) + pool
            out_ref[...] = jnp.dot(hh, w2_ref[...],
                                   preferred_element_type=jnp.float32) + b2_ref[...]

    return pl.pallas_call(
        body,
        grid=(nblk,),
        in_specs=[pl.BlockSpec((bn, D), lambda i: (i, 0)),
                  pl.BlockSpec((2, D), lambda i: (0, 0)),
                  pl.BlockSpec((1, 1, bn), lambda i: (i, 0, 0)),
                  pl.BlockSpec(fcW1.shape, lambda i: (0, 0)),
                  pl.BlockSpec((1, fcb1.shape[1]), lambda i: (0, 0)),
                  pl.BlockSpec(fcW2.shape, lambda i: (0, 0)),
                  pl.BlockSpec((1, 1), lambda i: (0, 0))],
        out_specs=pl.BlockSpec((_G, 1), lambda i: (0, 0)),
        out_shape=jax.ShapeDtypeStruct((_G, 1), jnp.float32),
        scratch_shapes=[pltpu.VMEM((_G, D), jnp.float32),
                        pltpu.VMEM((_G, 1), jnp.float32)],
    )(h, ss, batchr, fcW1, fcb1, fcW2, fcb2)


def kernel(x, edge_index, batch, eps1, W1a, b1a, W1b, b1b, g1, be1,
           eps2, W2a, b2a, W2b, b2b, g2, be2, fcW1, fcb1, fcW2, fcb2):
    N, F = x.shape
    E = edge_index.shape[1]
    D1 = W1a.shape[1]
    bn = 1000
    ch1, ch2 = 16, 128

    quantum = _NW * ch2 * _K
    epw = (-(-E // quantum) * quantum) // _NW
    epad = _NW * epw - E
    nch1, nch2 = epw // ch1, epw // ch2
    srcf = jnp.concatenate([edge_index[0], jnp.zeros((epad,), jnp.int32)])
    dstf = jnp.concatenate([edge_index[1], jnp.full((epad,), N, jnp.int32)])
    batchr = batch.reshape(N // bn, 1, bn)
    r2 = lambda a: a.reshape(1, -1)

    (p1,) = _segsum_sc(N, F, nch1, ch1, pre=False)(
        x, srcf.reshape(_NW, nch1, ch1), dstf.reshape(_NW, nch1, ch1))
    h1, ss1 = _conv1_post(x, p1, eps1.reshape(1, 1), W1a, r2(b1a), W1b,
                          r2(b1b), r2(g1), r2(be1), bn)
    x1d, p2 = _segsum_sc(N, D1, nch2, ch2, pre=True)(
        h1, ss1, srcf.reshape(_NW, nch2, ch2), dstf.reshape(_NW, nch2, ch2))
    h2, ss2 = _conv2_post(x1d[0], p2, eps2.reshape(1, 1), W2a, r2(b2a),
                          W2b, r2(b2b), r2(g2), r2(be2), bn)
    out = _pool_head(h2, ss2, batchr, fcW1, r2(fcb1), fcW2,
                     fcb2.reshape(1, 1), bn)
    return out.reshape(-1)


# speedup vs baseline: 1.0056x; 1.0056x over previous
import functools

import jax
import jax.numpy as jnp
from jax import lax
from jax.experimental import pallas as pl
from jax.experimental.pallas import tpu as pltpu
from jax.experimental.pallas import tpu_sc as plsc

_NC = 2
_NS = 16
_NW = _NC * _NS
_K = 8
_WB = 80
_G = 64


def _segsum_sc(N, D, nch, ch, pre):
    wb = _WB if D <= 32 else 40
    k = _K
    nwb = N // wb
    assert nwb * wb == N and D % 16 == 0 and nch % k == 0

    mesh = plsc.VectorSubcoreMesh(core_axis_name="c", subcore_axis_name="s",
                                  num_cores=_NC, num_subcores=_NS)
    out_type = [jax.ShapeDtypeStruct((_NC, N, D), jnp.float32)]
    if pre:
        out_type = [jax.ShapeDtypeStruct((_NC, N, D), jnp.float32)] + out_type
    scratch = [
        pltpu.VMEM((nch, ch), jnp.int32),
        pltpu.VMEM((nch, ch), jnp.int32),
        pltpu.VMEM((k, ch, D), jnp.float32),
        pltpu.VMEM((wb, D), jnp.float32),
        pltpu.VMEM_SHARED((N + 512, D), jnp.float32),
        pltpu.SemaphoreType.DMA,
        pltpu.SemaphoreType.DMA,
    ]
    if pre:
        scratch.insert(4, pltpu.VMEM((2, D), jnp.float32))

    @functools.partial(
        pl.kernel,
        out_type=out_type,
        mesh=mesh,
        compiler_params=pltpu.CompilerParams(use_tc_tiling_on_sc=False),
        scratch_types=scratch,
    )
    def seg(*refs):
        if pre:
            (h_hbm, ab_hbm, src_hbm, dst_hbm, x_hbm, out_hbm,
             src_v, dst_v, bufs, zbuf, ab_v, acc, gsem, ssem) = refs
        else:
            (y_hbm, src_hbm, dst_hbm, out_hbm,
             src_v, dst_v, bufs, zbuf, acc, gsem, ssem) = refs
        c = lax.axis_index("c")
        s = lax.axis_index("s")
        wid = c * _NS + s
        pltpu.sync_copy(src_hbm.at[wid], src_v)
        pltpu.sync_copy(dst_hbm.at[wid], dst_v)

        nrounds = pl.cdiv(nwb, _NS)

        if pre:
            pltpu.sync_copy(ab_hbm, ab_v)

            def achunk(kk, carry):
                chunk = s + kk * _NS

                @pl.when(chunk < nwb)
                def _():
                    pltpu.sync_copy(h_hbm.at[pl.ds(chunk * wb, wb)], zbuf)

                    def arow(r, c2):
                        for t in range(D // 16):
                            sl = pl.ds(t * 16, 16)
                            zbuf[r, sl] = jnp.maximum(
                                zbuf[r, sl] * ab_v[0, sl] + ab_v[1, sl], 0.0)
                        return c2

                    lax.fori_loop(0, wb, arow, 0)
                    pltpu.sync_copy(zbuf,
                                    x_hbm.at[c].at[pl.ds(chunk * wb, wb)])

                return carry

            lax.fori_loop(0, nrounds, achunk, 0)
            gather_src = x_hbm.at[c]
        else:
            gather_src = y_hbm

        zvec = jnp.zeros((16,), jnp.float32)

        def zrow(i, carry):
            for t in range(D // 16):
                zbuf[i, pl.ds(t * 16, 16)] = zvec
            return carry

        lax.fori_loop(0, wb, zrow, 0)

        def zchunk(kk, carry):
            chunk = s + kk * _NS

            @pl.when(chunk < nwb)
            def _():
                pltpu.sync_copy(zbuf, acc.at[pl.ds(chunk * wb, wb)])

            return carry

        lax.fori_loop(0, nrounds, zchunk, 0)
        plsc.subcore_barrier()

        def group(g, carry):
            gath = []
            for b in range(k):
                j = g * k + b
                gath.append(pltpu.async_copy(
                    gather_src.at[src_v.at[j]], bufs.at[b], gsem))
            for h in gath:
                h.wait()
            scat = []
            for b in range(k):
                j = g * k + b
                scat.append(pltpu.async_copy(
                    bufs.at[b], acc.at[dst_v.at[j]], ssem, add=True))
            for h in scat:
                h.wait()
            return carry

        lax.fori_loop(0, nch // k, group, 0)
        plsc.subcore_barrier()

        def wchunk(kk, carry):
            chunk = s + kk * _NS

            @pl.when(chunk < nwb)
            def _():
                pltpu.sync_copy(acc.at[pl.ds(chunk * wb, wb)],
                                out_hbm.at[c].at[pl.ds(chunk * wb, wb)])

            return carry

        lax.fori_loop(0, nrounds, wchunk, 0)

    return seg


def _conv1_post(x, p, eps, Wa, ba, Wb, bb, g, be, bn):
    N, F = x.shape
    D = Wb.shape[1]
    nblk = N // bn

    def body(x_ref, p_ref, eps_ref, wa_ref, ba_ref, wb_ref, bb_ref, g_ref,
             be_ref, h_ref, ss_ref, st_acc):
        i = pl.program_id(0)
        e = eps_ref[0, 0]
        pre = (1.0 + e) * x_ref[...] + jnp.sum(p_ref[...], axis=0)
        t = jax.nn.relu(jnp.dot(pre, wa_ref[...],
                                preferred_element_type=jnp.float32) + ba_ref[...])
        h = jnp.dot(t, wb_ref[...], preferred_element_type=jnp.float32) + bb_ref[...]
        h_ref[...] = h
        st = jnp.concatenate([jnp.sum(h, axis=0, keepdims=True),
                              jnp.sum(h * h, axis=0, keepdims=True)], axis=0)

        @pl.when(i == 0)
        def _():
            st_acc[...] = st

        @pl.when(i > 0)
        def _():
            st_acc[...] += st

        @pl.when(i == nblk - 1)
        def _():
            m = st_acc[0:1, :] * (1.0 / N)
            v = st_acc[1:2, :] * (1.0 / N) - m * m
            scale = lax.rsqrt(v + 1e-5) * g_ref[...]
            shift = be_ref[...] - m * scale
            ss_ref[...] = jnp.concatenate([scale, shift], axis=0)

    Da = Wa.shape[1]
    return pl.pallas_call(
        body,
        grid=(nblk,),
        in_specs=[pl.BlockSpec((bn, F), lambda i: (i, 0)),
                  pl.BlockSpec((2, bn, F), lambda i: (0, i, 0)),
                  pl.BlockSpec((1, 1), lambda i: (0, 0)),
                  pl.BlockSpec((F, Da), lambda i: (0, 0)),
                  pl.BlockSpec((1, Da), lambda i: (0, 0)),
                  pl.BlockSpec((Da, D), lambda i: (0, 0)),
                  pl.BlockSpec((1, D), lambda i: (0, 0)),
                  pl.BlockSpec((1, D), lambda i: (0, 0)),
                  pl.BlockSpec((1, D), lambda i: (0, 0))],
        out_specs=[pl.BlockSpec((bn, D), lambda i: (i, 0)),
                   pl.BlockSpec((2, D), lambda i: (0, 0))],
        out_shape=[jax.ShapeDtypeStruct((N, D), jnp.float32),
                   jax.ShapeDtypeStruct((2, D), jnp.float32)],
        scratch_shapes=[pltpu.VMEM((2, D), jnp.float32)],
    )(x, p, eps, Wa, ba, Wb, bb, g, be)


def _conv2_post(x1, p, eps, Wa, ba, Wb, bb, g, be, bn):
    N, D = x1.shape
    Do = Wb.shape[1]
    nblk = N // bn

    def body(x_ref, p_ref, eps_ref, wa_ref, ba_ref, wb_ref, bb_ref, g_ref,
             be_ref, h_ref, ss_ref, st_acc):
        i = pl.program_id(0)
        e = eps_ref[0, 0]
        pre = (1.0 + e) * x_ref[...] + jnp.sum(p_ref[...], axis=0)
        t = jax.nn.relu(jnp.dot(pre, wa_ref[...],
                                preferred_element_type=jnp.float32) + ba_ref[...])
        h = jnp.dot(t, wb_ref[...], preferred_element_type=jnp.float32) + bb_ref[...]
        h_ref[...] = h
        st = jnp.concatenate([jnp.sum(h, axis=0, keepdims=True),
                              jnp.sum(h * h, axis=0, keepdims=True)], axis=0)

        @pl.when(i == 0)
        def _():
            st_acc[...] = st

        @pl.when(i > 0)
        def _():
            st_acc[...] += st

        @pl.when(i == nblk - 1)
        def _():
            m = st_acc[0:1, :] * (1.0 / N)
            v = st_acc[1:2, :] * (1.0 / N) - m * m
            scale = lax.rsqrt(v + 1e-5) * g_ref[...]
            shift = be_ref[...] - m * scale
            ss_ref[...] = jnp.concatenate([scale, shift], axis=0)

    Da = Wa.shape[1]
    return pl.pallas_call(
        body,
        grid=(nblk,),
        in_specs=[pl.BlockSpec((bn, D), lambda i: (i, 0)),
                  pl.BlockSpec((2, bn, D), lambda i: (0, i, 0)),
                  pl.BlockSpec((1, 1), lambda i: (0, 0)),
                  pl.BlockSpec((D, Da), lambda i: (0, 0)),
                  pl.BlockSpec((1, Da), lambda i: (0, 0)),
                  pl.BlockSpec((Da, Do), lambda i: (0, 0)),
                  pl.BlockSpec((1, Do), lambda i: (0, 0)),
                  pl.BlockSpec((1, Do), lambda i: (0, 0)),
                  pl.BlockSpec((1, Do), lambda i: (0, 0))],
        out_specs=[pl.BlockSpec((bn, Do), lambda i: (i, 0)),
                   pl.BlockSpec((2, Do), lambda i: (0, 0))],
        out_shape=[jax.ShapeDtypeStruct((N, Do), jnp.float32),
                   jax.ShapeDtypeStruct((2, Do), jnp.float32)],
        scratch_shapes=[pltpu.VMEM((2, Do), jnp.float32)],
    )(x1, p, eps, Wa, ba, Wb, bb, g, be)


def _pool_head(h, ss, batchr, fcW1, fcb1, fcW2, fcb2, bn):
    N, D = h.shape
    nblk = N // bn

    def body(h_ref, ss_ref, b_ref, w1_ref, b1_ref, w2_ref, b2_ref,
             out_ref, pool_acc, cnt_acc):
        i = pl.program_id(0)
        x2 = jax.nn.relu(h_ref[...] * ss_ref[0:1, :] + ss_ref[1:2, :])
        oh = (lax.broadcasted_iota(jnp.int32, (_G, bn), 0)
              == b_ref[0]).astype(jnp.float32)
        pool_part = jnp.dot(oh, x2, preferred_element_type=jnp.float32,
                            precision=lax.Precision.HIGHEST)
        cnt_part = jnp.sum(oh, axis=1, keepdims=True)

        @pl.when(i == 0)
        def _():
            pool_acc[...] = pool_part
            cnt_acc[...] = cnt_part

        @pl.when(i > 0)
        def _():
            pool_acc[...] += pool_part
            cnt_acc[...] += cnt_part

        @pl.when(i == nblk - 1)
        def _():
            pool = pool_acc[...] / jnp.maximum(cnt_acc[...], 1.0)
            hh = jax.nn.relu(jnp.dot(pool, w1_ref[...],
                                     preferred_element_type=jnp.float32)
                             + b1_ref[...]) + pool
            out_ref[...] = jnp.dot(hh, w2_ref[...],
                                   preferred_element_type=jnp.float32) + b2_ref[...]

    return pl.pallas_call(
        body,
        grid=(nblk,),
        in_specs=[pl.BlockSpec((bn, D), lambda i: (i, 0)),
                  pl.BlockSpec((2, D), lambda i: (0, 0)),
                  pl.BlockSpec((1, 1, bn), lambda i: (i, 0, 0)),
                  pl.BlockSpec(fcW1.shape, lambda i: (0, 0)),
                  pl.BlockSpec((1, fcb1.shape[1]), lambda i: (0, 0)),
                  pl.BlockSpec(fcW2.shape, lambda i: (0, 0)),
                  pl.BlockSpec((1, 1), lambda i: (0, 0))],
        out_specs=pl.BlockSpec((_G, 1), lambda i: (0, 0)),
        out_shape=jax.ShapeDtypeStruct((_G, 1), jnp.float32),
        scratch_shapes=[pltpu.VMEM((_G, D), jnp.float32),
                        pltpu.VMEM((_G, 1), jnp.float32)],
    )(h, ss, batchr, fcW1, fcb1, fcW2, fcb2)


def kernel(x, edge_index, batch, eps1, W1a, b1a, W1b, b1b, g1, be1,
           eps2, W2a, b2a, W2b, b2b, g2, be2, fcW1, fcb1, fcW2, fcb2):
    N, F = x.shape
    E = edge_index.shape[1]
    D1 = W1a.shape[1]
    bn = 1000
    ch1, ch2 = 16, 128

    quantum = _NW * ch2 * _K
    epw = (-(-E // quantum) * quantum) // _NW
    epad = _NW * epw - E
    nch1, nch2 = epw // ch1, epw // ch2
    srcf = jnp.concatenate([edge_index[0], jnp.zeros((epad,), jnp.int32)])
    dstf = jnp.concatenate(
        [edge_index[1], N + (jnp.arange(epad, dtype=jnp.int32) % 512)])
    batchr = batch.reshape(N // bn, 1, bn)
    r2 = lambda a: a.reshape(1, -1)

    (p1,) = _segsum_sc(N, F, nch1, ch1, pre=False)(
        x, srcf.reshape(_NW, nch1, ch1), dstf.reshape(_NW, nch1, ch1))
    h1, ss1 = _conv1_post(x, p1, eps1.reshape(1, 1), W1a, r2(b1a), W1b,
                          r2(b1b), r2(g1), r2(be1), bn)
    x1d, p2 = _segsum_sc(N, D1, nch2, ch2, pre=True)(
        h1, ss1, srcf.reshape(_NW, nch2, ch2), dstf.reshape(_NW, nch2, ch2))
    h2, ss2 = _conv2_post(x1d[0], p2, eps2.reshape(1, 1), W2a, r2(b2a),
                          W2b, r2(b2b), r2(g2), r2(be2), bn)
    out = _pool_head(h2, ss2, batchr, fcW1, r2(fcb1), fcW2,
                     fcb2.reshape(1, 1), bn)
    return out.reshape(-1)
